# baseline (speedup 1.0000x reference)
import functools

import jax
import jax.numpy as jnp
import numpy as np
from jax import lax
from jax.experimental import pallas as pl
from jax.experimental.pallas import tpu as pltpu
from jax.experimental.pallas import tpu_sc as plsc

N = 10000
E = 320000
R = 20
H = 4
D = 128
NC = 2
NS = 16
NW = NC * NS
EW = E // NW
B = 80
T = EW // B
PC = 25
CH = 400
NCH = N // CH

BLK = 400
G = N // BLK

_f32 = jnp.float32

_S16_np = np.zeros((16, D), np.float32)
for _h in range(H):
    _S16_np[2 * _h, 32 * _h:32 * _h + 32] = 1.0

_P_np = np.zeros((H, 16), np.float32)
for _h in range(H):
    for _c in (2 * _h, 2 * _h + 1, 8 + 2 * _h, 8 + 2 * _h + 1):
        _P_np[_h, _c] = 1.0


def _transform_body(h, w_ref, as_ref, ad_ref, l_ref, wh_ref, ss_ref, sd_ref, sl_ref):
    for r in range(R):
        wh = jnp.dot(h, w_ref[r], preferred_element_type=_f32)
        wh_ref[r] = wh
        ss_ref[r] = jnp.dot(wh, as_ref[...], preferred_element_type=_f32)
        sd_ref[r] = jnp.dot(wh, ad_ref[...], preferred_element_type=_f32)
    sl_ref[...] = jnp.dot(h, l_ref[...], preferred_element_type=_f32)


def _tc_first(x, W, A_s, A_d, L):
    def body(x_ref, w_ref, as_ref, ad_ref, l_ref, wh_ref, ss_ref, sd_ref, sl_ref):
        _transform_body(x_ref[...], w_ref, as_ref, ad_ref, l_ref,
                        wh_ref, ss_ref, sd_ref, sl_ref)

    return pl.pallas_call(
        body,
        grid=(G,),
        in_specs=[
            pl.BlockSpec((BLK, D), lambda i: (i, 0)),
            pl.BlockSpec((R, D, D), lambda i: (0, 0, 0)),
            pl.BlockSpec((D, 16), lambda i: (0, 0)),
            pl.BlockSpec((D, 16), lambda i: (0, 0)),
            pl.BlockSpec((D, D), lambda i: (0, 0)),
        ],
        out_specs=[
            pl.BlockSpec((R, BLK, D), lambda i: (0, i, 0)),
            pl.BlockSpec((R, BLK, 16), lambda i: (0, i, 0)),
            pl.BlockSpec((R, BLK, 16), lambda i: (0, i, 0)),
            pl.BlockSpec((BLK, D), lambda i: (i, 0)),
        ],
        out_shape=[
            jax.ShapeDtypeStruct((R, N, D), _f32),
            jax.ShapeDtypeStruct((R, N, 16), _f32),
            jax.ShapeDtypeStruct((R, N, 16), _f32),
            jax.ShapeDtypeStruct((N, D), _f32),
        ],
    )(x, W, A_s, A_d, L)


def _epilogue_h(acc0_ref, acc1_ref, den0_ref, den1_ref, sl_ref, s_ref):
    den = den0_ref[...] + den1_ref[...]
    dbc = jnp.dot(den, s_ref[...], preferred_element_type=_f32) + 1e-9
    return (acc0_ref[...] + acc1_ref[...]) / dbc + sl_ref[...]


def _tc_mid(acc, den, sl, W, A_s, A_d, L, S16):
    def body(acc0_ref, acc1_ref, den0_ref, den1_ref, sl_ref, s_ref,
             w_ref, as_ref, ad_ref, l_ref, wh_ref, ss_ref, sd_ref, slo_ref):
        h = jnp.maximum(_epilogue_h(acc0_ref, acc1_ref, den0_ref, den1_ref,
                                    sl_ref, s_ref), 0.0)
        _transform_body(h, w_ref, as_ref, ad_ref, l_ref,
                        wh_ref, ss_ref, sd_ref, slo_ref)

    return pl.pallas_call(
        body,
        grid=(G,),
        in_specs=[
            pl.BlockSpec((BLK, D), lambda i: (i, 0)),
            pl.BlockSpec((BLK, D), lambda i: (i, 0)),
            pl.BlockSpec((BLK, 16), lambda i: (i, 0)),
            pl.BlockSpec((BLK, 16), lambda i: (i, 0)),
            pl.BlockSpec((BLK, D), lambda i: (i, 0)),
            pl.BlockSpec((16, D), lambda i: (0, 0)),
            pl.BlockSpec((R, D, D), lambda i: (0, 0, 0)),
            pl.BlockSpec((D, 16), lambda i: (0, 0)),
            pl.BlockSpec((D, 16), lambda i: (0, 0)),
            pl.BlockSpec((D, D), lambda i: (0, 0)),
        ],
        out_specs=[
            pl.BlockSpec((R, BLK, D), lambda i: (0, i, 0)),
            pl.BlockSpec((R, BLK, 16), lambda i: (0, i, 0)),
            pl.BlockSpec((R, BLK, 16), lambda i: (0, i, 0)),
            pl.BlockSpec((BLK, D), lambda i: (i, 0)),
        ],
        out_shape=[
            jax.ShapeDtypeStruct((R, N, D), _f32),
            jax.ShapeDtypeStruct((R, N, 16), _f32),
            jax.ShapeDtypeStruct((R, N, 16), _f32),
            jax.ShapeDtypeStruct((N, D), _f32),
        ],
    )(acc[0], acc[1], den[0], den[1], sl, S16, W, A_s, A_d, L)


def _tc_final(acc, den, sl, S16):
    def body(acc0_ref, acc1_ref, den0_ref, den1_ref, sl_ref, s_ref, out_ref):
        out_ref[...] = _epilogue_h(acc0_ref, acc1_ref, den0_ref, den1_ref,
                                   sl_ref, s_ref)

    return pl.pallas_call(
        body,
        grid=(G,),
        in_specs=[
            pl.BlockSpec((BLK, D), lambda i: (i, 0)),
            pl.BlockSpec((BLK, D), lambda i: (i, 0)),
            pl.BlockSpec((BLK, 16), lambda i: (i, 0)),
            pl.BlockSpec((BLK, 16), lambda i: (i, 0)),
            pl.BlockSpec((BLK, D), lambda i: (i, 0)),
            pl.BlockSpec((16, D), lambda i: (0, 0)),
        ],
        out_specs=pl.BlockSpec((BLK, D), lambda i: (i, 0)),
        out_shape=jax.ShapeDtypeStruct((N, D), _f32),
    )(acc[0], acc[1], den[0], den[1], sl, S16)


_mesh = plsc.VectorSubcoreMesh(core_axis_name="c", subcore_axis_name="s",
                               num_cores=NC, num_subcores=NS)

_GD = lax.GatherDimensionNumbers(
    offset_dims=(), collapsed_slice_dims=(0,), start_index_map=(0,))


@functools.partial(
    pl.kernel,
    mesh=_mesh,
    compiler_params=pltpu.CompilerParams(use_tc_tiling_on_sc=False),
    out_type=(
        jax.ShapeDtypeStruct((NC, N, D), _f32),
        jax.ShapeDtypeStruct((NC, N, 16), _f32),
    ),
    scratch_types=[
        pltpu.VMEM((PC, 3, B), jnp.int32),
        pltpu.VMEM((B, 16), _f32),
        pltpu.VMEM((B, 16), _f32),
        pltpu.VMEM((B, 16), _f32),
        pltpu.VMEM((B, 16), _f32),
        pltpu.VMEM((B, 16), _f32),
        pltpu.VMEM((B, 16), _f32),
        pltpu.VMEM((B, D), _f32),
        pltpu.VMEM((B, D), _f32),
        pltpu.VMEM_SHARED((N, D), _f32),
        pltpu.VMEM_SHARED((N, 16), _f32),
        pltpu.SemaphoreType.DMA,
        pltpu.SemaphoreType.DMA,
        pltpu.SemaphoreType.DMA,
        pltpu.SemaphoreType.DMA,
    ],
)
def _sc_edges(wh_hbm, ss_hbm, sd_hbm, pidx_hbm,
              z128_hbm, z16_hbm, acc_out, den_out,
              pidx_all, ls0, ls1, ld0, ld1, ex0, ex1, rows0, rows1,
              acc_sh, den_sh, gsem0, gsem1, ssem0, ssem1):
    c = lax.axis_index("c")
    s = lax.axis_index("s")
    wid = s * NC + c
    ls = (ls0, ls1)
    ld = (ld0, ld1)
    exb = (ex0, ex1)
    rows = (rows0, rows1)
    gsem = (gsem0, gsem1)
    ssem = (ssem0, ssem1)

    def _zero_chunk(ci):
        off = ci * CH
        pltpu.sync_copy(z128_hbm.at[pl.ds(off, CH)], acc_sh.at[pl.ds(off, CH)])
        pltpu.sync_copy(z16_hbm.at[pl.ds(off, CH)], den_sh.at[pl.ds(off, CH)])

    _zero_chunk(s)
    pl.when(s < NCH - NS)(lambda: _zero_chunk(s + NS))
    plsc.subcore_barrier()

    def gath(i, k):
        pltpu.async_copy(ss_hbm.at[pidx_all.at[i, 0]], ls[k], gsem[k])
        pltpu.async_copy(sd_hbm.at[pidx_all.at[i, 1]], ld[k], gsem[k])
        pltpu.async_copy(wh_hbm.at[pidx_all.at[i, 0]], rows[k], gsem[k])

    def wait_gath(i, k):
        pltpu.make_async_copy(ss_hbm.at[pidx_all.at[i, 0]], ls[k], gsem[k]).wait()
        pltpu.make_async_copy(sd_hbm.at[pidx_all.at[i, 1]], ld[k], gsem[k]).wait()
        pltpu.make_async_copy(wh_hbm.at[pidx_all.at[i, 0]], rows[k], gsem[k]).wait()

    def scat(i, k):
        pltpu.async_copy(exb[k], den_sh.at[pidx_all.at[i, 2]], ssem[k], add=True)
        pltpu.async_copy(rows[k], acc_sh.at[pidx_all.at[i, 2]], ssem[k], add=True)

    def wait_scat(i, k):
        pltpu.make_async_copy(exb[k], den_sh.at[pidx_all.at[i, 2]], ssem[k]).wait()
        pltpu.make_async_copy(rows[k], acc_sh.at[pidx_all.at[i, 2]], ssem[k]).wait()

    def compute(k):
        lsk, ldk, exk, rk = ls[k], ld[k], exb[k], rows[k]

        @plsc.parallel_loop(0, B, unroll=8)
        def edge(b):
            lg = lsk[b, :] + ldk[b, :]
            ex = jnp.exp(jnp.maximum(lg, lg * 0.2))
            exk[b, :] = ex
            for h4 in range(H):
                m = lax.gather(
                    ex, jnp.full((16, 1), 2 * h4, jnp.int32),
                    dimension_numbers=_GD, slice_sizes=(1,),
                    mode=lax.GatherScatterMode.PROMISE_IN_BOUNDS)
                for j in (2 * h4, 2 * h4 + 1):
                    rk[b, pl.ds(16 * j, 16)] = rk[b, pl.ds(16 * j, 16)] * m

    for f in range(T // PC):
        pltpu.sync_copy(pidx_hbm.at[pl.ds(wid * T + f * PC, PC)], pidx_all)
        gath(0, 0)

        def pair(ob, carry):
            for k in (0, 1):
                j = ob * 2 + k
                kn = 1 - k
                wait_gath(j, k)
                pl.when(j >= 1)(lambda: wait_scat(j - 1, kn))
                gath(j + 1, kn)
                compute(k)
                scat(j, k)
            return carry

        lax.fori_loop(0, (PC - 1) // 2, pair, 0)
        wait_gath(PC - 1, 0)
        compute(0)
        scat(PC - 1, 0)
        wait_scat(PC - 2, 1)
        wait_scat(PC - 1, 0)
    plsc.subcore_barrier()

    def _out_chunk(ci):
        off = ci * CH
        pltpu.sync_copy(acc_sh.at[pl.ds(off, CH)],
                        acc_out.at[c, pl.ds(off, CH)])
        pltpu.sync_copy(den_sh.at[pl.ds(off, CH)],
                        den_out.at[c, pl.ds(off, CH)])

    _out_chunk(s)
    pl.when(s < NCH - NS)(lambda: _out_chunk(s + NS))


def _score_mat(a):
    return jnp.einsum('ho,hc->hoc', a, jnp.asarray(_P_np)).reshape(D, 16)


def kernel(x, edge_index, edge_type,
           W1, a1s, a1d, L1,
           W2, a2s, a2d, L2,
           W3, a3s, a3d, L3,
           W4, a4s, a4d, L4):
    src = edge_index[0]
    dst = edge_index[1]
    et = edge_type
    isrc = et * N + src
    idst = et * N + dst
    pidx = jnp.stack([isrc.reshape(NW, T, B), idst.reshape(NW, T, B),
                      dst.reshape(NW, T, B)], axis=2).reshape(NW * T, 3, B)
    z128 = jnp.zeros((N, D), _f32)
    z16 = jnp.zeros((N, 16), _f32)
    S16 = jnp.asarray(_S16_np)

    cols = np.concatenate([np.arange(20) + 32 * h for h in range(H)])
    W4p = jnp.zeros((R, D, D), _f32).at[:, :, cols].set(W4)
    L4p = jnp.zeros((D, D), _f32).at[:, cols].set(L4)
    a4sp = jnp.zeros((H, 32), _f32).at[:, :20].set(a4s)
    a4dp = jnp.zeros((H, 32), _f32).at[:, :20].set(a4d)

    layers = [
        (W1, a1s, a1d, L1),
        (W2, a2s, a2d, L2),
        (W3, a3s, a3d, L3),
        (W4p, a4sp, a4dp, L4p),
    ]

    sl = None
    acc = den = None
    for li, (W, a_s, a_d, L) in enumerate(layers):
        A_s, A_d = _score_mat(a_s), _score_mat(a_d)
        if li == 0:
            wh, ss, sd, sl = _tc_first(x, W, A_s, A_d, L)
        else:
            wh, ss, sd, sl = _tc_mid(acc, den, sl, W, A_s, A_d, L, S16)
        acc, den = _sc_edges(wh.reshape(R * N, D), ss.reshape(R * N, 16),
                             sd.reshape(R * N, 16), pidx, z128, z16)
    out = _tc_final(acc, den, sl, S16)
    return jnp.concatenate([out[:, 32 * k:32 * k + 20] for k in range(H)], axis=1)


# speedup vs baseline: 1.0057x; 1.0057x over previous
import functools

import jax
import jax.numpy as jnp
import numpy as np
from jax import lax
from jax.experimental import pallas as pl
from jax.experimental.pallas import tpu as pltpu
from jax.experimental.pallas import tpu_sc as plsc

N = 10000
E = 320000
R = 20
H = 4
D = 128
NC = 2
NS = 16
NW = NC * NS
EW = E // NW
B = 80
T = EW // B
PC = 25
CH = 400
NCH = N // CH

BLK = 400
G = N // BLK

_f32 = jnp.float32

_S16_np = np.zeros((16, D), np.float32)
for _h in range(H):
    _S16_np[2 * _h, 32 * _h:32 * _h + 32] = 1.0

_P_np = np.zeros((H, 16), np.float32)
for _h in range(H):
    for _c in (2 * _h, 2 * _h + 1, 8 + 2 * _h, 8 + 2 * _h + 1):
        _P_np[_h, _c] = 1.0


def _transform_body(h, w_ref, as_ref, ad_ref, l_ref, wh_ref, ss_ref, sd_ref, sl_ref):
    for r in range(R):
        wh = jnp.dot(h, w_ref[r], preferred_element_type=_f32)
        wh_ref[r] = wh
        ss_ref[r] = jnp.dot(wh, as_ref[...], preferred_element_type=_f32)
        sd_ref[r] = jnp.dot(wh, ad_ref[...], preferred_element_type=_f32)
    sl_ref[...] = jnp.dot(h, l_ref[...], preferred_element_type=_f32)


def _tc_first(x, W, A_s, A_d, L):
    def body(x_ref, w_ref, as_ref, ad_ref, l_ref, wh_ref, ss_ref, sd_ref, sl_ref):
        _transform_body(x_ref[...], w_ref, as_ref, ad_ref, l_ref,
                        wh_ref, ss_ref, sd_ref, sl_ref)

    return pl.pallas_call(
        body,
        grid=(G,),
        in_specs=[
            pl.BlockSpec((BLK, D), lambda i: (i, 0)),
            pl.BlockSpec((R, D, D), lambda i: (0, 0, 0)),
            pl.BlockSpec((D, 16), lambda i: (0, 0)),
            pl.BlockSpec((D, 16), lambda i: (0, 0)),
            pl.BlockSpec((D, D), lambda i: (0, 0)),
        ],
        out_specs=[
            pl.BlockSpec((R, BLK, D), lambda i: (0, i, 0)),
            pl.BlockSpec((R, BLK, 16), lambda i: (0, i, 0)),
            pl.BlockSpec((R, BLK, 16), lambda i: (0, i, 0)),
            pl.BlockSpec((BLK, D), lambda i: (i, 0)),
        ],
        out_shape=[
            jax.ShapeDtypeStruct((R, N, D), _f32),
            jax.ShapeDtypeStruct((R, N, 16), _f32),
            jax.ShapeDtypeStruct((R, N, 16), _f32),
            jax.ShapeDtypeStruct((N, D), _f32),
        ],
    )(x, W, A_s, A_d, L)


def _epilogue_h(acc0_ref, acc1_ref, den0_ref, den1_ref, sl_ref, s_ref):
    den = den0_ref[...] + den1_ref[...]
    dbc = jnp.dot(den, s_ref[...], preferred_element_type=_f32) + 1e-9
    return (acc0_ref[...] + acc1_ref[...]) / dbc + sl_ref[...]


def _tc_mid(acc, den, sl, W, A_s, A_d, L, S16):
    def body(acc0_ref, acc1_ref, den0_ref, den1_ref, sl_ref, s_ref,
             w_ref, as_ref, ad_ref, l_ref, wh_ref, ss_ref, sd_ref, slo_ref):
        h = jnp.maximum(_epilogue_h(acc0_ref, acc1_ref, den0_ref, den1_ref,
                                    sl_ref, s_ref), 0.0)
        _transform_body(h, w_ref, as_ref, ad_ref, l_ref,
                        wh_ref, ss_ref, sd_ref, slo_ref)

    return pl.pallas_call(
        body,
        grid=(G,),
        in_specs=[
            pl.BlockSpec((BLK, D), lambda i: (i, 0)),
            pl.BlockSpec((BLK, D), lambda i: (i, 0)),
            pl.BlockSpec((BLK, 16), lambda i: (i, 0)),
            pl.BlockSpec((BLK, 16), lambda i: (i, 0)),
            pl.BlockSpec((BLK, D), lambda i: (i, 0)),
            pl.BlockSpec((16, D), lambda i: (0, 0)),
            pl.BlockSpec((R, D, D), lambda i: (0, 0, 0)),
            pl.BlockSpec((D, 16), lambda i: (0, 0)),
            pl.BlockSpec((D, 16), lambda i: (0, 0)),
            pl.BlockSpec((D, D), lambda i: (0, 0)),
        ],
        out_specs=[
            pl.BlockSpec((R, BLK, D), lambda i: (0, i, 0)),
            pl.BlockSpec((R, BLK, 16), lambda i: (0, i, 0)),
            pl.BlockSpec((R, BLK, 16), lambda i: (0, i, 0)),
            pl.BlockSpec((BLK, D), lambda i: (i, 0)),
        ],
        out_shape=[
            jax.ShapeDtypeStruct((R, N, D), _f32),
            jax.ShapeDtypeStruct((R, N, 16), _f32),
            jax.ShapeDtypeStruct((R, N, 16), _f32),
            jax.ShapeDtypeStruct((N, D), _f32),
        ],
    )(acc[0], acc[1], den[0], den[1], sl, S16, W, A_s, A_d, L)


def _tc_final(acc, den, sl, S16):
    def body(acc0_ref, acc1_ref, den0_ref, den1_ref, sl_ref, s_ref, out_ref):
        out_ref[...] = _epilogue_h(acc0_ref, acc1_ref, den0_ref, den1_ref,
                                   sl_ref, s_ref)

    return pl.pallas_call(
        body,
        grid=(G,),
        in_specs=[
            pl.BlockSpec((BLK, D), lambda i: (i, 0)),
            pl.BlockSpec((BLK, D), lambda i: (i, 0)),
            pl.BlockSpec((BLK, 16), lambda i: (i, 0)),
            pl.BlockSpec((BLK, 16), lambda i: (i, 0)),
            pl.BlockSpec((BLK, D), lambda i: (i, 0)),
            pl.BlockSpec((16, D), lambda i: (0, 0)),
        ],
        out_specs=pl.BlockSpec((BLK, D), lambda i: (i, 0)),
        out_shape=jax.ShapeDtypeStruct((N, D), _f32),
    )(acc[0], acc[1], den[0], den[1], sl, S16)


_mesh = plsc.VectorSubcoreMesh(core_axis_name="c", subcore_axis_name="s",
                               num_cores=NC, num_subcores=NS)

_GD = lax.GatherDimensionNumbers(
    offset_dims=(), collapsed_slice_dims=(0,), start_index_map=(0,))


@functools.partial(
    pl.kernel,
    mesh=_mesh,
    compiler_params=pltpu.CompilerParams(use_tc_tiling_on_sc=False),
    out_type=(
        jax.ShapeDtypeStruct((NC, N, D), _f32),
        jax.ShapeDtypeStruct((NC, N, 16), _f32),
    ),
    scratch_types=[
        pltpu.VMEM((PC, 3, B), jnp.int32),
        pltpu.VMEM((B, 16), _f32),
        pltpu.VMEM((B, 16), _f32),
        pltpu.VMEM((B, 16), _f32),
        pltpu.VMEM((B, 16), _f32),
        pltpu.VMEM((B, 16), _f32),
        pltpu.VMEM((B, 16), _f32),
        pltpu.VMEM((B, D), _f32),
        pltpu.VMEM((B, D), _f32),
        pltpu.VMEM_SHARED((N, D), _f32),
        pltpu.VMEM_SHARED((N, 16), _f32),
        pltpu.SemaphoreType.DMA,
        pltpu.SemaphoreType.DMA,
        pltpu.SemaphoreType.DMA,
        pltpu.SemaphoreType.DMA,
    ],
)
def _sc_edges(wh_hbm, ss_hbm, sd_hbm, pidx_hbm,
              z128_hbm, z16_hbm, acc_out, den_out,
              pidx_all, ls0, ls1, ld0, ld1, ex0, ex1, rows0, rows1,
              acc_sh, den_sh, gsem0, gsem1, ssem0, ssem1):
    c = lax.axis_index("c")
    s = lax.axis_index("s")
    wid = s * NC + c
    ls = (ls0, ls1)
    ld = (ld0, ld1)
    exb = (ex0, ex1)
    rows = (rows0, rows1)
    gsem = (gsem0, gsem1)
    ssem = (ssem0, ssem1)

    def _zero_chunk(ci):
        off = ci * CH
        pltpu.sync_copy(z128_hbm.at[pl.ds(off, CH)], acc_sh.at[pl.ds(off, CH)])
        pltpu.sync_copy(z16_hbm.at[pl.ds(off, CH)], den_sh.at[pl.ds(off, CH)])

    _zero_chunk(s)
    pl.when(s < NCH - NS)(lambda: _zero_chunk(s + NS))
    plsc.subcore_barrier()

    def gath(i, k):
        pltpu.async_copy(ss_hbm.at[pidx_all.at[i, 0]], ls[k], gsem[k])
        pltpu.async_copy(sd_hbm.at[pidx_all.at[i, 1]], ld[k], gsem[k])
        pltpu.async_copy(wh_hbm.at[pidx_all.at[i, 0]], rows[k], gsem[k])

    def wait_gath(i, k):
        pltpu.make_async_copy(ss_hbm.at[pidx_all.at[i, 0]], ls[k], gsem[k]).wait()
        pltpu.make_async_copy(sd_hbm.at[pidx_all.at[i, 1]], ld[k], gsem[k]).wait()
        pltpu.make_async_copy(wh_hbm.at[pidx_all.at[i, 0]], rows[k], gsem[k]).wait()

    def scat(i, k):
        pltpu.async_copy(exb[k], den_sh.at[pidx_all.at[i, 2]], ssem[k], add=True)
        pltpu.async_copy(rows[k], acc_sh.at[pidx_all.at[i, 2]], ssem[k], add=True)

    def wait_scat(i, k):
        pltpu.make_async_copy(exb[k], den_sh.at[pidx_all.at[i, 2]], ssem[k]).wait()
        pltpu.make_async_copy(rows[k], acc_sh.at[pidx_all.at[i, 2]], ssem[k]).wait()

    def compute(k):
        lsk, ldk, exk, rk = ls[k], ld[k], exb[k], rows[k]

        @plsc.parallel_loop(0, B, unroll=4)
        def edge(b):
            lg = lsk[b, :] + ldk[b, :]
            ex = jnp.exp(jnp.maximum(lg, lg * 0.2))
            exk[b, :] = ex
            for h4 in range(H):
                m = lax.gather(
                    ex, jnp.full((16, 1), 2 * h4, jnp.int32),
                    dimension_numbers=_GD, slice_sizes=(1,),
                    mode=lax.GatherScatterMode.PROMISE_IN_BOUNDS)
                for j in (2 * h4, 2 * h4 + 1):
                    rk[b, pl.ds(16 * j, 16)] = rk[b, pl.ds(16 * j, 16)] * m

    for f in range(T // PC):
        pltpu.sync_copy(pidx_hbm.at[pl.ds(wid * T + f * PC, PC)], pidx_all)
        gath(0, 0)

        def pair(ob, carry):
            for k in (0, 1):
                j = ob * 2 + k
                kn = 1 - k
                wait_gath(j, k)
                pl.when(j >= 1)(lambda: wait_scat(j - 1, kn))
                gath(j + 1, kn)
                compute(k)
                scat(j, k)
            return carry

        lax.fori_loop(0, (PC - 1) // 2, pair, 0)
        wait_gath(PC - 1, 0)
        compute(0)
        scat(PC - 1, 0)
        wait_scat(PC - 2, 1)
        wait_scat(PC - 1, 0)
    plsc.subcore_barrier()

    def _out_chunk(ci):
        off = ci * CH
        pltpu.sync_copy(acc_sh.at[pl.ds(off, CH)],
                        acc_out.at[c, pl.ds(off, CH)])
        pltpu.sync_copy(den_sh.at[pl.ds(off, CH)],
                        den_out.at[c, pl.ds(off, CH)])

    _out_chunk(s)
    pl.when(s < NCH - NS)(lambda: _out_chunk(s + NS))


def _score_mat(a):
    return jnp.einsum('ho,hc->hoc', a, jnp.asarray(_P_np)).reshape(D, 16)


def kernel(x, edge_index, edge_type,
           W1, a1s, a1d, L1,
           W2, a2s, a2d, L2,
           W3, a3s, a3d, L3,
           W4, a4s, a4d, L4):
    src = edge_index[0]
    dst = edge_index[1]
    et = edge_type
    isrc = et * N + src
    idst = et * N + dst
    pidx = jnp.stack([isrc.reshape(NW, T, B), idst.reshape(NW, T, B),
                      dst.reshape(NW, T, B)], axis=2).reshape(NW * T, 3, B)
    z128 = jnp.zeros((N, D), _f32)
    z16 = jnp.zeros((N, 16), _f32)
    S16 = jnp.asarray(_S16_np)

    cols = np.concatenate([np.arange(20) + 32 * h for h in range(H)])
    W4p = jnp.zeros((R, D, D), _f32).at[:, :, cols].set(W4)
    L4p = jnp.zeros((D, D), _f32).at[:, cols].set(L4)
    a4sp = jnp.zeros((H, 32), _f32).at[:, :20].set(a4s)
    a4dp = jnp.zeros((H, 32), _f32).at[:, :20].set(a4d)

    layers = [
        (W1, a1s, a1d, L1),
        (W2, a2s, a2d, L2),
        (W3, a3s, a3d, L3),
        (W4p, a4sp, a4dp, L4p),
    ]

    sl = None
    acc = den = None
    for li, (W, a_s, a_d, L) in enumerate(layers):
        A_s, A_d = _score_mat(a_s), _score_mat(a_d)
        if li == 0:
            wh, ss, sd, sl = _tc_first(x, W, A_s, A_d, L)
        else:
            wh, ss, sd, sl = _tc_mid(acc, den, sl, W, A_s, A_d, L, S16)
        acc, den = _sc_edges(wh.reshape(R * N, D), ss.reshape(R * N, 16),
                             sd.reshape(R * N, 16), pidx, z128, z16)
    out = _tc_final(acc, den, sl, S16)
    return jnp.concatenate([out[:, 32 * k:32 * k + 20] for k in range(H)], axis=1)


# speedup vs baseline: 1.0299x; 1.0241x over previous
import functools

import jax
import jax.numpy as jnp
import numpy as np
from jax import lax
from jax.experimental import pallas as pl
from jax.experimental.pallas import tpu as pltpu
from jax.experimental.pallas import tpu_sc as plsc

N = 10000
E = 320000
R = 20
H = 4
D = 128
NC = 2
NS = 16
NW = NC * NS
EW = E // NW
B = 80
T = EW // B
PC = 25
CH = 400
NCH = N // CH

BLK = 400
G = N // BLK

_f32 = jnp.float32

_S16_np = np.zeros((16, D), np.float32)
for _h in range(H):
    _S16_np[2 * _h, 32 * _h:32 * _h + 32] = 1.0

_P_np = np.zeros((H, 16), np.float32)
for _h in range(H):
    for _c in (2 * _h, 2 * _h + 1, 8 + 2 * _h, 8 + 2 * _h + 1):
        _P_np[_h, _c] = 1.0


def _transform_body(h, w_ref, l_ref, wh_ref, ss_ref, sd_ref, sl_ref):
    for r in range(R):
        out = jnp.dot(h, w_ref[r], preferred_element_type=_f32)
        wh_ref[r] = out[:, 0:D]
        ss_ref[r] = out[:, D:D + 16]
        sd_ref[r] = out[:, D + 16:D + 32]
    sl_ref[...] = jnp.dot(h, l_ref[...], preferred_element_type=_f32)


def _tc_first(x, Wcat, L):
    def body(x_ref, w_ref, l_ref, wh_ref, ss_ref, sd_ref, sl_ref):
        _transform_body(x_ref[...], w_ref, l_ref,
                        wh_ref, ss_ref, sd_ref, sl_ref)

    return pl.pallas_call(
        body,
        grid=(G,),
        in_specs=[
            pl.BlockSpec((BLK, D), lambda i: (i, 0)),
            pl.BlockSpec((R, D, D + 32), lambda i: (0, 0, 0)),
            pl.BlockSpec((D, D), lambda i: (0, 0)),
        ],
        out_specs=[
            pl.BlockSpec((R, BLK, D), lambda i: (0, i, 0)),
            pl.BlockSpec((R, BLK, 16), lambda i: (0, i, 0)),
            pl.BlockSpec((R, BLK, 16), lambda i: (0, i, 0)),
            pl.BlockSpec((BLK, D), lambda i: (i, 0)),
        ],
        out_shape=[
            jax.ShapeDtypeStruct((R, N, D), _f32),
            jax.ShapeDtypeStruct((R, N, 16), _f32),
            jax.ShapeDtypeStruct((R, N, 16), _f32),
            jax.ShapeDtypeStruct((N, D), _f32),
        ],
    )(x, Wcat, L)


def _epilogue_h(acc0_ref, acc1_ref, den0_ref, den1_ref, sl_ref, s_ref):
    den = den0_ref[...] + den1_ref[...]
    dbc = jnp.dot(den, s_ref[...], preferred_element_type=_f32) + 1e-9
    return (acc0_ref[...] + acc1_ref[...]) / dbc + sl_ref[...]


def _tc_mid(acc, den, sl, Wcat, L, S16):
    def body(acc0_ref, acc1_ref, den0_ref, den1_ref, sl_ref, s_ref,
             w_ref, l_ref, wh_ref, ss_ref, sd_ref, slo_ref):
        h = jnp.maximum(_epilogue_h(acc0_ref, acc1_ref, den0_ref, den1_ref,
                                    sl_ref, s_ref), 0.0)
        _transform_body(h, w_ref, l_ref, wh_ref, ss_ref, sd_ref, slo_ref)

    return pl.pallas_call(
        body,
        grid=(G,),
        in_specs=[
            pl.BlockSpec((BLK, D), lambda i: (i, 0)),
            pl.BlockSpec((BLK, D), lambda i: (i, 0)),
            pl.BlockSpec((BLK, 16), lambda i: (i, 0)),
            pl.BlockSpec((BLK, 16), lambda i: (i, 0)),
            pl.BlockSpec((BLK, D), lambda i: (i, 0)),
            pl.BlockSpec((16, D), lambda i: (0, 0)),
            pl.BlockSpec((R, D, D + 32), lambda i: (0, 0, 0)),
            pl.BlockSpec((D, D), lambda i: (0, 0)),
        ],
        out_specs=[
            pl.BlockSpec((R, BLK, D), lambda i: (0, i, 0)),
            pl.BlockSpec((R, BLK, 16), lambda i: (0, i, 0)),
            pl.BlockSpec((R, BLK, 16), lambda i: (0, i, 0)),
            pl.BlockSpec((BLK, D), lambda i: (i, 0)),
        ],
        out_shape=[
            jax.ShapeDtypeStruct((R, N, D), _f32),
            jax.ShapeDtypeStruct((R, N, 16), _f32),
            jax.ShapeDtypeStruct((R, N, 16), _f32),
            jax.ShapeDtypeStruct((N, D), _f32),
        ],
    )(acc[0], acc[1], den[0], den[1], sl, S16, Wcat, L)


def _tc_final(acc, den, sl, S16):
    def body(acc0_ref, acc1_ref, den0_ref, den1_ref, sl_ref, s_ref, out_ref):
        out_ref[...] = _epilogue_h(acc0_ref, acc1_ref, den0_ref, den1_ref,
                                   sl_ref, s_ref)

    return pl.pallas_call(
        body,
        grid=(G,),
        in_specs=[
            pl.BlockSpec((BLK, D), lambda i: (i, 0)),
            pl.BlockSpec((BLK, D), lambda i: (i, 0)),
            pl.BlockSpec((BLK, 16), lambda i: (i, 0)),
            pl.BlockSpec((BLK, 16), lambda i: (i, 0)),
            pl.BlockSpec((BLK, D), lambda i: (i, 0)),
            pl.BlockSpec((16, D), lambda i: (0, 0)),
        ],
        out_specs=pl.BlockSpec((BLK, D), lambda i: (i, 0)),
        out_shape=jax.ShapeDtypeStruct((N, D), _f32),
    )(acc[0], acc[1], den[0], den[1], sl, S16)


_mesh = plsc.VectorSubcoreMesh(core_axis_name="c", subcore_axis_name="s",
                               num_cores=NC, num_subcores=NS)

_GD = lax.GatherDimensionNumbers(
    offset_dims=(), collapsed_slice_dims=(0,), start_index_map=(0,))


@functools.partial(
    pl.kernel,
    mesh=_mesh,
    compiler_params=pltpu.CompilerParams(use_tc_tiling_on_sc=False),
    out_type=(
        jax.ShapeDtypeStruct((NC, N, D), _f32),
        jax.ShapeDtypeStruct((NC, N, 16), _f32),
    ),
    scratch_types=[
        pltpu.VMEM((PC, 3, B), jnp.int32),
        pltpu.VMEM((B, 16), _f32),
        pltpu.VMEM((B, 16), _f32),
        pltpu.VMEM((B, 16), _f32),
        pltpu.VMEM((B, 16), _f32),
        pltpu.VMEM((B, 16), _f32),
        pltpu.VMEM((B, 16), _f32),
        pltpu.VMEM((B, D), _f32),
        pltpu.VMEM((B, D), _f32),
        pltpu.VMEM_SHARED((N, D), _f32),
        pltpu.VMEM_SHARED((N, 16), _f32),
        pltpu.SemaphoreType.DMA,
        pltpu.SemaphoreType.DMA,
        pltpu.SemaphoreType.DMA,
        pltpu.SemaphoreType.DMA,
    ],
)
def _sc_edges(wh_hbm, ss_hbm, sd_hbm, pidx_hbm,
              z128_hbm, z16_hbm, acc_out, den_out,
              pidx_all, ls0, ls1, ld0, ld1, ex0, ex1, rows0, rows1,
              acc_sh, den_sh, gsem0, gsem1, ssem0, ssem1):
    c = lax.axis_index("c")
    s = lax.axis_index("s")
    wid = s * NC + c
    ls = (ls0, ls1)
    ld = (ld0, ld1)
    exb = (ex0, ex1)
    rows = (rows0, rows1)
    gsem = (gsem0, gsem1)
    ssem = (ssem0, ssem1)

    def _zero_chunk(ci):
        off = ci * CH
        pltpu.sync_copy(z128_hbm.at[pl.ds(off, CH)], acc_sh.at[pl.ds(off, CH)])
        pltpu.sync_copy(z16_hbm.at[pl.ds(off, CH)], den_sh.at[pl.ds(off, CH)])

    _zero_chunk(s)
    pl.when(s < NCH - NS)(lambda: _zero_chunk(s + NS))
    plsc.subcore_barrier()

    def gath(i, k):
        pltpu.async_copy(ss_hbm.at[pidx_all.at[i, 0]], ls[k], gsem[k])
        pltpu.async_copy(sd_hbm.at[pidx_all.at[i, 1]], ld[k], gsem[k])
        pltpu.async_copy(wh_hbm.at[pidx_all.at[i, 0]], rows[k], gsem[k])

    def wait_gath(i, k):
        pltpu.make_async_copy(ss_hbm.at[pidx_all.at[i, 0]], ls[k], gsem[k]).wait()
        pltpu.make_async_copy(sd_hbm.at[pidx_all.at[i, 1]], ld[k], gsem[k]).wait()
        pltpu.make_async_copy(wh_hbm.at[pidx_all.at[i, 0]], rows[k], gsem[k]).wait()

    def scat(i, k):
        pltpu.async_copy(exb[k], den_sh.at[pidx_all.at[i, 2]], ssem[k], add=True)
        pltpu.async_copy(rows[k], acc_sh.at[pidx_all.at[i, 2]], ssem[k], add=True)

    def wait_scat(i, k):
        pltpu.make_async_copy(exb[k], den_sh.at[pidx_all.at[i, 2]], ssem[k]).wait()
        pltpu.make_async_copy(rows[k], acc_sh.at[pidx_all.at[i, 2]], ssem[k]).wait()

    def compute(k):
        lsk, ldk, exk, rk = ls[k], ld[k], exb[k], rows[k]

        @plsc.parallel_loop(0, B, unroll=4)
        def edge(b):
            lg = lsk[b, :] + ldk[b, :]
            ex = jnp.exp(jnp.maximum(lg, lg * 0.2))
            exk[b, :] = ex
            for h4 in range(H):
                m = lax.gather(
                    ex, jnp.full((16, 1), 2 * h4, jnp.int32),
                    dimension_numbers=_GD, slice_sizes=(1,),
                    mode=lax.GatherScatterMode.PROMISE_IN_BOUNDS)
                for j in (2 * h4, 2 * h4 + 1):
                    rk[b, pl.ds(16 * j, 16)] = rk[b, pl.ds(16 * j, 16)] * m

    for f in range(T // PC):
        pltpu.sync_copy(pidx_hbm.at[pl.ds(wid * T + f * PC, PC)], pidx_all)
        gath(0, 0)

        def pair(ob, carry):
            for k in (0, 1):
                j = ob * 2 + k
                kn = 1 - k
                wait_gath(j, k)
                pl.when(j >= 1)(lambda: wait_scat(j - 1, kn))
                gath(j + 1, kn)
                compute(k)
                scat(j, k)
            return carry

        lax.fori_loop(0, (PC - 1) // 2, pair, 0)
        wait_gath(PC - 1, 0)
        compute(0)
        scat(PC - 1, 0)
        wait_scat(PC - 2, 1)
        wait_scat(PC - 1, 0)
    plsc.subcore_barrier()

    def _out_chunk(ci):
        off = ci * CH
        pltpu.sync_copy(acc_sh.at[pl.ds(off, CH)],
                        acc_out.at[c, pl.ds(off, CH)])
        pltpu.sync_copy(den_sh.at[pl.ds(off, CH)],
                        den_out.at[c, pl.ds(off, CH)])

    _out_chunk(s)
    pl.when(s < NCH - NS)(lambda: _out_chunk(s + NS))


def _score_mat(a):
    return jnp.einsum('ho,hc->hoc', a, jnp.asarray(_P_np)).reshape(D, 16)


def kernel(x, edge_index, edge_type,
           W1, a1s, a1d, L1,
           W2, a2s, a2d, L2,
           W3, a3s, a3d, L3,
           W4, a4s, a4d, L4):
    src = edge_index[0]
    dst = edge_index[1]
    et = edge_type
    isrc = et * N + src
    idst = et * N + dst
    pidx = jnp.stack([isrc.reshape(NW, T, B), idst.reshape(NW, T, B),
                      dst.reshape(NW, T, B)], axis=2).reshape(NW * T, 3, B)
    z128 = jnp.zeros((N, D), _f32)
    z16 = jnp.zeros((N, 16), _f32)
    S16 = jnp.asarray(_S16_np)

    cols = np.concatenate([np.arange(20) + 32 * h for h in range(H)])
    W4p = jnp.zeros((R, D, D), _f32).at[:, :, cols].set(W4)
    L4p = jnp.zeros((D, D), _f32).at[:, cols].set(L4)
    a4sp = jnp.zeros((H, 32), _f32).at[:, :20].set(a4s)
    a4dp = jnp.zeros((H, 32), _f32).at[:, :20].set(a4d)

    layers = [
        (W1, a1s, a1d, L1),
        (W2, a2s, a2d, L2),
        (W3, a3s, a3d, L3),
        (W4p, a4sp, a4dp, L4p),
    ]

    sl = None
    acc = den = None
    for li, (W, a_s, a_d, L) in enumerate(layers):
        A_s, A_d = _score_mat(a_s), _score_mat(a_d)
        Wcat = jnp.concatenate(
            [W, jnp.einsum('rdo,oc->rdc', W, A_s),
             jnp.einsum('rdo,oc->rdc', W, A_d)], axis=2)
        if li == 0:
            wh, ss, sd, sl = _tc_first(x, Wcat, L)
        else:
            wh, ss, sd, sl = _tc_mid(acc, den, sl, Wcat, L, S16)
        acc, den = _sc_edges(wh.reshape(R * N, D), ss.reshape(R * N, 16),
                             sd.reshape(R * N, 16), pidx, z128, z16)
    out = _tc_final(acc, den, sl, S16)
    return jnp.concatenate([out[:, 32 * k:32 * k + 20] for k in range(H)], axis=1)
